# Initial kernel scaffold; baseline (speedup 1.0000x reference)
import jax
import jax.numpy as jnp
from jax.experimental import pallas as pl


def kernel(node_feat, coord, virtual_node_feat, virtual_coord, We1, be1, We2, be2, Wc1, bc1, Wc2, Wn1, bn1, Wn2, bn2, data_batch):
    raise NotImplementedError("write your pallas kernel here")


# speedup vs baseline: 6.8420x; 6.8420x over previous
import functools

import jax
import jax.numpy as jnp
from jax.experimental import pallas as pl
from jax.experimental.pallas import tpu as pltpu


def _silu(x):
    return x * jax.nn.sigmoid(x)


def _egcl_block_kernel(nf_ref, crd_ref, db_ref, vnf_ref, vc_ref,
                       we1n_ref, we1v_ref, we1r_ref, be1_ref,
                       we2_ref, be2_ref, wc1_ref, bc1_ref, wc2_ref,
                       wn1a_ref, wn1b_ref, bn1_ref, wn2_ref, bn2_ref,
                       outf_ref, outc_ref,
                       accn_ref, accc_ref, vscr_ref,
                       *, nb, c_dim, h_dim):
    i = pl.program_id(0)
    b_dim = vnf_ref.shape[0]
    s = nf_ref.shape[0]
    ch = c_dim * h_dim
    f32 = jnp.float32

    @pl.when(i == 0)
    def _init():
        accn_ref[...] = jnp.zeros_like(accn_ref)
        accc_ref[...] = jnp.zeros_like(accc_ref)
        for c in range(c_dim):
            vscr_ref[:, c * h_dim:(c + 1) * h_dim] = jnp.dot(
                vnf_ref[:, c * h_dim:(c + 1) * h_dim], we1v_ref[...],
                preferred_element_type=f32)
        vscr_ref[:, ch:ch + 3 * c_dim] = vc_ref[...]

    db = db_ref[0, 0, :]
    onehot = (db[:, None] == jax.lax.broadcasted_iota(
        jnp.int32, (s, b_dim), 1)).astype(f32)
    onehot_t = (db[None, :] == jax.lax.broadcasted_iota(
        jnp.int32, (b_dim, s), 0)).astype(f32)

    gat = jnp.dot(onehot, vscr_ref[...], preferred_element_type=f32)

    crd = crd_ref[...]
    crd_rep = jnp.concatenate(
        [crd[:, k:k + 1] for k in range(3) for _ in range(c_dim)], axis=1)
    d12 = gat[:, ch:ch + 3 * c_dim] - crd_rep

    nfw = jnp.dot(nf_ref[...], we1n_ref[...], preferred_element_type=f32)

    be1 = be1_ref[...]
    we1r = we1r_ref[...]
    svals = []
    for c in range(c_dim):
        rad2 = (d12[:, c:c + 1] ** 2 + d12[:, c_dim + c:c_dim + c + 1] ** 2
                + d12[:, 2 * c_dim + c:2 * c_dim + c + 1] ** 2)
        radial = jnp.sqrt(rad2)
        pre1 = nfw + gat[:, c * h_dim:(c + 1) * h_dim] + radial * we1r + be1
        h1 = _silu(pre1)
        h2 = _silu(jnp.dot(h1, we2_ref[...], preferred_element_type=f32)
                   + be2_ref[...])
        part = jnp.dot(onehot_t, h2, preferred_element_type=f32)
        accn_ref[:, c * h_dim:(c + 1) * h_dim] += part
        cm = _silu(jnp.dot(h2, wc1_ref[...], preferred_element_type=f32)
                   + bc1_ref[...])
        svals.append(jnp.dot(cm, wc2_ref[...], preferred_element_type=f32))

    s12 = jnp.concatenate(svals * 3, axis=1)
    ones = jnp.ones((s, 1), dtype=f32)
    trans = jnp.concatenate([d12 * s12, ones], axis=1)
    accc_ref[...] += jnp.dot(onehot_t, trans, preferred_element_type=f32)

    @pl.when(i == nb - 1)
    def _epilogue():
        cnt = jnp.maximum(accc_ref[:, 3 * c_dim:3 * c_dim + 1], 1.0)
        inv = 1.0 / cnt
        outc_ref[...] = vc_ref[...] + accc_ref[:, :3 * c_dim] * inv
        for c in range(c_dim):
            vnf_c = vnf_ref[:, c * h_dim:(c + 1) * h_dim]
            agg_c = accn_ref[:, c * h_dim:(c + 1) * h_dim] * inv
            t = _silu(jnp.dot(vnf_c, wn1a_ref[...], preferred_element_type=f32)
                      + jnp.dot(agg_c, wn1b_ref[...], preferred_element_type=f32)
                      + bn1_ref[...])
            o = (jnp.dot(t, wn2_ref[...], preferred_element_type=f32)
                 + bn2_ref[...] + vnf_c)
            outf_ref[:, c * h_dim:(c + 1) * h_dim] = o


def kernel(node_feat, coord, virtual_node_feat, virtual_coord,
           We1, be1, We2, be2, Wc1, bc1, Wc2, Wn1, bn1, Wn2, bn2,
           data_batch):
    n, nf = node_feat.shape
    b, h, c = virtual_node_feat.shape
    ch = c * h

    s = 2000
    nb = -(-n // s)
    n_pad = nb * s
    if n_pad != n:
        node_feat = jnp.pad(node_feat, ((0, n_pad - n), (0, 0)))
        coord = jnp.pad(coord, ((0, n_pad - n), (0, 0)))
        data_batch = jnp.pad(data_batch, (0, n_pad - n),
                             constant_values=jnp.int32(b))

    vnf_t = virtual_node_feat.transpose(0, 2, 1).reshape(b, ch)
    vc_flat = virtual_coord.reshape(b, 3 * c)
    db3 = data_batch.reshape(nb, 1, s)

    we1_nf = We1[:nf]
    we1_v = We1[nf:nf + h]
    we1_r = We1[2 * nf:2 * nf + 1]
    wn1a = Wn1[:h]
    wn1b = Wn1[h:]

    row = lambda v: v.reshape(1, -1)

    grid_spec = pltpu.PrefetchScalarGridSpec(
        num_scalar_prefetch=0,
        grid=(nb,),
        in_specs=[
            pl.BlockSpec((s, nf), lambda i: (i, 0)),
            pl.BlockSpec((s, 3), lambda i: (i, 0)),
            pl.BlockSpec((1, 1, s), lambda i: (i, 0, 0)),
            pl.BlockSpec((b, ch), lambda i: (0, 0)),
            pl.BlockSpec((b, 3 * c), lambda i: (0, 0)),
            pl.BlockSpec((nf, h), lambda i: (0, 0)),
            pl.BlockSpec((h, h), lambda i: (0, 0)),
            pl.BlockSpec((1, h), lambda i: (0, 0)),
            pl.BlockSpec((1, h), lambda i: (0, 0)),
            pl.BlockSpec((h, h), lambda i: (0, 0)),
            pl.BlockSpec((1, h), lambda i: (0, 0)),
            pl.BlockSpec((h, h), lambda i: (0, 0)),
            pl.BlockSpec((1, h), lambda i: (0, 0)),
            pl.BlockSpec((h, 1), lambda i: (0, 0)),
            pl.BlockSpec((h, h), lambda i: (0, 0)),
            pl.BlockSpec((h, h), lambda i: (0, 0)),
            pl.BlockSpec((1, h), lambda i: (0, 0)),
            pl.BlockSpec((h, h), lambda i: (0, 0)),
            pl.BlockSpec((1, h), lambda i: (0, 0)),
        ],
        out_specs=[
            pl.BlockSpec((b, ch), lambda i: (0, 0)),
            pl.BlockSpec((b, 3 * c), lambda i: (0, 0)),
        ],
        scratch_shapes=[
            pltpu.VMEM((b, ch), jnp.float32),
            pltpu.VMEM((b, 3 * c + 1), jnp.float32),
            pltpu.VMEM((b, ch + 3 * c), jnp.float32),
        ],
    )

    outf, outc = pl.pallas_call(
        functools.partial(_egcl_block_kernel, nb=nb, c_dim=c, h_dim=h),
        grid_spec=grid_spec,
        out_shape=[
            jax.ShapeDtypeStruct((b, ch), jnp.float32),
            jax.ShapeDtypeStruct((b, 3 * c), jnp.float32),
        ],
        compiler_params=pltpu.CompilerParams(
            dimension_semantics=("arbitrary",)),
    )(node_feat, coord, db3, vnf_t, vc_flat,
      we1_nf, we1_v, we1_r, row(be1), We2, row(be2),
      Wc1, row(bc1), Wc2, wn1a, wn1b, row(bn1), Wn2, row(bn2))

    new_vfeat = outf.reshape(b, c, h).transpose(0, 2, 1)
    new_vcoord = outc.reshape(b, 3, c)
    return (new_vfeat, new_vcoord)


# speedup vs baseline: 7.9626x; 1.1638x over previous
import functools

import jax
import jax.numpy as jnp
from jax.experimental import pallas as pl
from jax.experimental.pallas import tpu as pltpu


def _silu(x):
    return x * jax.nn.sigmoid(x)


def _egcl_block_kernel(nf_ref, crd_ref, db_ref, vnf_ref, vc_ref,
                       we1n_ref, we1v_ref, we1r_ref, be1_ref,
                       we2_ref, be2_ref, wc1_ref, bc1_ref, wc2_ref,
                       wn1a_ref, wn1b_ref, bn1_ref, wn2_ref, bn2_ref,
                       sel_ref,
                       outf_ref, outc_ref,
                       accn_ref, accc_ref, vscr_ref,
                       *, nb, c_dim, h_dim):
    i = pl.program_id(0)
    b_dim = vnf_ref.shape[0]
    s = nf_ref.shape[0]
    ch = c_dim * h_dim
    f32 = jnp.float32
    bf16 = jnp.bfloat16

    @pl.when(i == 0)
    def _init():
        accn_ref[...] = jnp.zeros_like(accn_ref)
        accc_ref[...] = jnp.zeros_like(accc_ref)
        for c in range(c_dim):
            vscr_ref[:, c * h_dim:(c + 1) * h_dim] = jnp.dot(
                vnf_ref[:, c * h_dim:(c + 1) * h_dim], we1v_ref[...],
                preferred_element_type=f32).astype(bf16)
        vscr_ref[:, ch:ch + 3 * c_dim] = vc_ref[...].astype(bf16)

    db = db_ref[0, 0, :]
    onehot = (db[:, None] == jax.lax.broadcasted_iota(
        jnp.int32, (s, b_dim), 1)).astype(bf16)
    onehot_t = (db[None, :] == jax.lax.broadcasted_iota(
        jnp.int32, (b_dim, s), 0)).astype(bf16)

    gat = jnp.dot(onehot, vscr_ref[...], preferred_element_type=f32)

    crd = crd_ref[...]
    crd_rep = jnp.concatenate(
        [crd[:, k:k + 1] for k in range(3) for _ in range(c_dim)], axis=1)
    d12 = gat[:, ch:ch + 3 * c_dim] - crd_rep

    rad_all = jnp.sqrt(jnp.dot((d12 * d12).astype(bf16), sel_ref[...],
                               preferred_element_type=f32))

    nfw = jnp.dot(nf_ref[...].astype(bf16), we1n_ref[...].astype(bf16),
                  preferred_element_type=f32)

    be1 = be1_ref[...]
    we1r = we1r_ref[...]
    we2 = we2_ref[...].astype(bf16)
    wc1 = wc1_ref[...].astype(bf16)
    wc2 = wc2_ref[...].astype(bf16)
    svals = []
    for c in range(c_dim):
        pre1 = (nfw + gat[:, c * h_dim:(c + 1) * h_dim]
                + rad_all[:, c:c + 1] * we1r + be1)
        h1 = _silu(pre1).astype(bf16)
        h2 = _silu(jnp.dot(h1, we2, preferred_element_type=f32)
                   + be2_ref[...])
        h2b = h2.astype(bf16)
        part = jnp.dot(onehot_t, h2b, preferred_element_type=f32)
        accn_ref[:, c * h_dim:(c + 1) * h_dim] += part
        cm = _silu(jnp.dot(h2b, wc1, preferred_element_type=f32)
                   + bc1_ref[...]).astype(bf16)
        svals.append(jnp.dot(cm, wc2, preferred_element_type=f32))

    s4 = jnp.concatenate(svals, axis=1)
    s12 = jnp.concatenate([s4, s4, s4], axis=1)
    ones = jnp.ones((s, 1), dtype=bf16)
    trans = jnp.concatenate([(d12 * s12).astype(bf16), ones], axis=1)
    accc_ref[...] += jnp.dot(onehot_t, trans, preferred_element_type=f32)

    @pl.when(i == nb - 1)
    def _epilogue():
        cnt = jnp.maximum(accc_ref[:, 3 * c_dim:3 * c_dim + 1], 1.0)
        inv = 1.0 / cnt
        outc_ref[...] = vc_ref[...] + accc_ref[:, :3 * c_dim] * inv
        for c in range(c_dim):
            vnf_c = vnf_ref[:, c * h_dim:(c + 1) * h_dim]
            agg_c = accn_ref[:, c * h_dim:(c + 1) * h_dim] * inv
            t = _silu(jnp.dot(vnf_c, wn1a_ref[...], preferred_element_type=f32)
                      + jnp.dot(agg_c, wn1b_ref[...], preferred_element_type=f32)
                      + bn1_ref[...])
            o = (jnp.dot(t, wn2_ref[...], preferred_element_type=f32)
                 + bn2_ref[...] + vnf_c)
            outf_ref[:, c * h_dim:(c + 1) * h_dim] = o


def kernel(node_feat, coord, virtual_node_feat, virtual_coord,
           We1, be1, We2, be2, Wc1, bc1, Wc2, Wn1, bn1, Wn2, bn2,
           data_batch):
    n, nf = node_feat.shape
    b, h, c = virtual_node_feat.shape
    ch = c * h

    s = 2000
    nb = -(-n // s)
    n_pad = nb * s
    if n_pad != n:
        node_feat = jnp.pad(node_feat, ((0, n_pad - n), (0, 0)))
        coord = jnp.pad(coord, ((0, n_pad - n), (0, 0)))
        data_batch = jnp.pad(data_batch, (0, n_pad - n),
                             constant_values=jnp.int32(b))

    vnf_t = virtual_node_feat.transpose(0, 2, 1).reshape(b, ch)
    vc_flat = virtual_coord.reshape(b, 3 * c)
    db3 = data_batch.reshape(nb, 1, s)

    we1_nf = We1[:nf]
    we1_v = We1[nf:nf + h]
    we1_r = We1[2 * nf:2 * nf + 1]
    wn1a = Wn1[:h]
    wn1b = Wn1[h:]

    row = lambda v: v.reshape(1, -1)

    grid_spec = pltpu.PrefetchScalarGridSpec(
        num_scalar_prefetch=0,
        grid=(nb,),
        in_specs=[
            pl.BlockSpec((s, nf), lambda i: (i, 0)),
            pl.BlockSpec((s, 3), lambda i: (i, 0)),
            pl.BlockSpec((1, 1, s), lambda i: (i, 0, 0)),
            pl.BlockSpec((b, ch), lambda i: (0, 0)),
            pl.BlockSpec((b, 3 * c), lambda i: (0, 0)),
            pl.BlockSpec((nf, h), lambda i: (0, 0)),
            pl.BlockSpec((h, h), lambda i: (0, 0)),
            pl.BlockSpec((1, h), lambda i: (0, 0)),
            pl.BlockSpec((1, h), lambda i: (0, 0)),
            pl.BlockSpec((h, h), lambda i: (0, 0)),
            pl.BlockSpec((1, h), lambda i: (0, 0)),
            pl.BlockSpec((h, h), lambda i: (0, 0)),
            pl.BlockSpec((1, h), lambda i: (0, 0)),
            pl.BlockSpec((h, 1), lambda i: (0, 0)),
            pl.BlockSpec((h, h), lambda i: (0, 0)),
            pl.BlockSpec((h, h), lambda i: (0, 0)),
            pl.BlockSpec((1, h), lambda i: (0, 0)),
            pl.BlockSpec((h, h), lambda i: (0, 0)),
            pl.BlockSpec((1, h), lambda i: (0, 0)),
            pl.BlockSpec((3 * c, c), lambda i: (0, 0)),
        ],
        out_specs=[
            pl.BlockSpec((b, ch), lambda i: (0, 0)),
            pl.BlockSpec((b, 3 * c), lambda i: (0, 0)),
        ],
        scratch_shapes=[
            pltpu.VMEM((b, ch), jnp.float32),
            pltpu.VMEM((b, 3 * c + 1), jnp.float32),
            pltpu.VMEM((b, ch + 3 * c), jnp.bfloat16),
        ],
    )

    call = pl.pallas_call(
        functools.partial(_egcl_block_kernel, nb=nb, c_dim=c, h_dim=h),
        grid_spec=grid_spec,
        out_shape=[
            jax.ShapeDtypeStruct((b, ch), jnp.float32),
            jax.ShapeDtypeStruct((b, 3 * c), jnp.float32),
        ],
        compiler_params=pltpu.CompilerParams(
            dimension_semantics=("arbitrary",)),
    )

    sel = jnp.tile(jnp.eye(c, dtype=jnp.bfloat16), (3, 1))

    outf, outc = call(node_feat, coord, db3, vnf_t, vc_flat,
                      we1_nf, we1_v, we1_r, row(be1), We2, row(be2),
                      Wc1, row(bc1), Wc2, wn1a, wn1b, row(bn1), Wn2,
                      row(bn2), sel)

    new_vfeat = outf.reshape(b, c, h).transpose(0, 2, 1)
    new_vcoord = outc.reshape(b, 3, c)
    return (new_vfeat, new_vcoord)


# speedup vs baseline: 8.0769x; 1.0144x over previous
import functools

import jax
import jax.numpy as jnp
from jax.experimental import pallas as pl
from jax.experimental.pallas import tpu as pltpu


def _silu(x):
    return 0.5 * x * (jnp.tanh(0.5 * x) + 1.0)


def _egcl_block_kernel(nf_ref, crd_ref, db_ref, vnf_ref, vc_ref,
                       we1n_ref, we1v_ref, we1r_ref, be1_ref,
                       we2_ref, be2_ref, wc1_ref, bc1_ref, wc2_ref,
                       wn1a_ref, wn1b_ref, bn1_ref, wn2_ref, bn2_ref,
                       sel_ref,
                       outf_ref, outc_ref,
                       accn_ref, accc_ref, vscr_ref,
                       *, nb, c_dim, h_dim):
    i = pl.program_id(0)
    b_dim = vnf_ref.shape[0]
    s = nf_ref.shape[0]
    ch = c_dim * h_dim
    f32 = jnp.float32
    bf16 = jnp.bfloat16

    @pl.when(i == 0)
    def _init():
        accn_ref[...] = jnp.zeros_like(accn_ref)
        accc_ref[...] = jnp.zeros_like(accc_ref)
        for c in range(c_dim):
            vscr_ref[:, c * h_dim:(c + 1) * h_dim] = jnp.dot(
                vnf_ref[:, c * h_dim:(c + 1) * h_dim], we1v_ref[...],
                preferred_element_type=f32).astype(bf16)
        vscr_ref[:, ch:ch + 3 * c_dim] = vc_ref[...].astype(bf16)

    db = db_ref[0, 0, :]
    onehot = (db[:, None] == jax.lax.broadcasted_iota(
        jnp.int32, (s, b_dim), 1)).astype(bf16)
    onehot_t = (db[None, :] == jax.lax.broadcasted_iota(
        jnp.int32, (b_dim, s), 0)).astype(bf16)

    gat = jnp.dot(onehot, vscr_ref[...], preferred_element_type=f32)

    crd = crd_ref[...]
    crd_rep = jnp.concatenate(
        [crd[:, k:k + 1] for k in range(3) for _ in range(c_dim)], axis=1)
    d12 = gat[:, ch:ch + 3 * c_dim] - crd_rep

    rad_all = jnp.sqrt(jnp.dot((d12 * d12).astype(bf16), sel_ref[...],
                               preferred_element_type=f32))

    nfw = jnp.dot(nf_ref[...].astype(bf16), we1n_ref[...].astype(bf16),
                  preferred_element_type=f32)

    be1 = be1_ref[...]
    we1r = we1r_ref[...]
    we2 = we2_ref[...].astype(bf16)
    wc1 = wc1_ref[...].astype(bf16)
    wc2 = wc2_ref[...].astype(bf16)
    svals = []
    for c in range(c_dim):
        pre1 = (nfw + gat[:, c * h_dim:(c + 1) * h_dim]
                + rad_all[:, c:c + 1] * we1r + be1)
        h1 = _silu(pre1).astype(bf16)
        h2 = _silu(jnp.dot(h1, we2, preferred_element_type=f32)
                   + be2_ref[...])
        h2b = h2.astype(bf16)
        part = jnp.dot(onehot_t, h2b, preferred_element_type=f32)
        accn_ref[:, c * h_dim:(c + 1) * h_dim] += part
        cm = _silu(jnp.dot(h2b, wc1, preferred_element_type=f32)
                   + bc1_ref[...]).astype(bf16)
        svals.append(jnp.dot(cm, wc2, preferred_element_type=f32))

    s4 = jnp.concatenate(svals, axis=1)
    s12 = jnp.concatenate([s4, s4, s4], axis=1)
    ones = jnp.ones((s, 1), dtype=bf16)
    trans = jnp.concatenate([(d12 * s12).astype(bf16), ones], axis=1)
    accc_ref[...] += jnp.dot(onehot_t, trans, preferred_element_type=f32)

    @pl.when(i == nb - 1)
    def _epilogue():
        cnt = jnp.maximum(accc_ref[:, 3 * c_dim:3 * c_dim + 1], 1.0)
        inv = 1.0 / cnt
        outc_ref[...] = vc_ref[...] + accc_ref[:, :3 * c_dim] * inv
        for c in range(c_dim):
            vnf_c = vnf_ref[:, c * h_dim:(c + 1) * h_dim]
            agg_c = accn_ref[:, c * h_dim:(c + 1) * h_dim] * inv
            t = _silu(jnp.dot(vnf_c, wn1a_ref[...], preferred_element_type=f32)
                      + jnp.dot(agg_c, wn1b_ref[...], preferred_element_type=f32)
                      + bn1_ref[...])
            o = (jnp.dot(t, wn2_ref[...], preferred_element_type=f32)
                 + bn2_ref[...] + vnf_c)
            outf_ref[:, c * h_dim:(c + 1) * h_dim] = o


def kernel(node_feat, coord, virtual_node_feat, virtual_coord,
           We1, be1, We2, be2, Wc1, bc1, Wc2, Wn1, bn1, Wn2, bn2,
           data_batch):
    n, nf = node_feat.shape
    b, h, c = virtual_node_feat.shape
    ch = c * h

    s = 2000
    nb = -(-n // s)
    n_pad = nb * s
    if n_pad != n:
        node_feat = jnp.pad(node_feat, ((0, n_pad - n), (0, 0)))
        coord = jnp.pad(coord, ((0, n_pad - n), (0, 0)))
        data_batch = jnp.pad(data_batch, (0, n_pad - n),
                             constant_values=jnp.int32(b))

    vnf_t = virtual_node_feat.transpose(0, 2, 1).reshape(b, ch)
    vc_flat = virtual_coord.reshape(b, 3 * c)
    db3 = data_batch.reshape(nb, 1, s)

    we1_nf = We1[:nf]
    we1_v = We1[nf:nf + h]
    we1_r = We1[2 * nf:2 * nf + 1]
    wn1a = Wn1[:h]
    wn1b = Wn1[h:]

    row = lambda v: v.reshape(1, -1)

    grid_spec = pltpu.PrefetchScalarGridSpec(
        num_scalar_prefetch=0,
        grid=(nb,),
        in_specs=[
            pl.BlockSpec((s, nf), lambda i: (i, 0)),
            pl.BlockSpec((s, 3), lambda i: (i, 0)),
            pl.BlockSpec((1, 1, s), lambda i: (i, 0, 0)),
            pl.BlockSpec((b, ch), lambda i: (0, 0)),
            pl.BlockSpec((b, 3 * c), lambda i: (0, 0)),
            pl.BlockSpec((nf, h), lambda i: (0, 0)),
            pl.BlockSpec((h, h), lambda i: (0, 0)),
            pl.BlockSpec((1, h), lambda i: (0, 0)),
            pl.BlockSpec((1, h), lambda i: (0, 0)),
            pl.BlockSpec((h, h), lambda i: (0, 0)),
            pl.BlockSpec((1, h), lambda i: (0, 0)),
            pl.BlockSpec((h, h), lambda i: (0, 0)),
            pl.BlockSpec((1, h), lambda i: (0, 0)),
            pl.BlockSpec((h, 1), lambda i: (0, 0)),
            pl.BlockSpec((h, h), lambda i: (0, 0)),
            pl.BlockSpec((h, h), lambda i: (0, 0)),
            pl.BlockSpec((1, h), lambda i: (0, 0)),
            pl.BlockSpec((h, h), lambda i: (0, 0)),
            pl.BlockSpec((1, h), lambda i: (0, 0)),
            pl.BlockSpec((3 * c, c), lambda i: (0, 0)),
        ],
        out_specs=[
            pl.BlockSpec((b, ch), lambda i: (0, 0)),
            pl.BlockSpec((b, 3 * c), lambda i: (0, 0)),
        ],
        scratch_shapes=[
            pltpu.VMEM((b, ch), jnp.float32),
            pltpu.VMEM((b, 3 * c + 1), jnp.float32),
            pltpu.VMEM((b, ch + 3 * c), jnp.bfloat16),
        ],
    )

    call = pl.pallas_call(
        functools.partial(_egcl_block_kernel, nb=nb, c_dim=c, h_dim=h),
        grid_spec=grid_spec,
        out_shape=[
            jax.ShapeDtypeStruct((b, ch), jnp.float32),
            jax.ShapeDtypeStruct((b, 3 * c), jnp.float32),
        ],
        compiler_params=pltpu.CompilerParams(
            dimension_semantics=("arbitrary",)),
    )

    sel = jnp.tile(jnp.eye(c, dtype=jnp.bfloat16), (3, 1))

    outf, outc = call(node_feat, coord, db3, vnf_t, vc_flat,
                      we1_nf, we1_v, we1_r, row(be1), We2, row(be2),
                      Wc1, row(bc1), Wc2, wn1a, wn1b, row(bn1), Wn2,
                      row(bn2), sel)

    new_vfeat = outf.reshape(b, c, h).transpose(0, 2, 1)
    new_vcoord = outc.reshape(b, 3, c)
    return (new_vfeat, new_vcoord)


# speedup vs baseline: 9.8466x; 1.2191x over previous
import functools

import jax
import jax.numpy as jnp
from jax.experimental import pallas as pl
from jax.experimental.pallas import tpu as pltpu


def _silu(x):
    return 0.5 * x * (jnp.tanh(0.5 * x) + 1.0)


def _egcl_block_kernel(nf_ref, crd_ref, db_ref, vnf_ref, vc_ref,
                       we1n_ref, we1v_ref, we1r_ref, be1_ref,
                       we2_ref, be2_ref, wc1_ref, bc1_ref, wc2_ref,
                       wn1a_ref, wn1b_ref, bn1_ref, wn2_ref, bn2_ref,
                       sel_ref, wc2blk_ref,
                       outf_ref, outc_ref,
                       accn_ref, accc_ref, vscr_ref,
                       *, nb, c_dim, h_dim):
    i = pl.program_id(0)
    b_dim = vnf_ref.shape[0]
    s = nf_ref.shape[0]
    ch = c_dim * h_dim
    f32 = jnp.float32
    bf16 = jnp.bfloat16

    @pl.when(i == 0)
    def _init():
        accn_ref[...] = jnp.zeros_like(accn_ref)
        accc_ref[...] = jnp.zeros_like(accc_ref)
        for c in range(c_dim):
            vscr_ref[:, c * h_dim:(c + 1) * h_dim] = (jnp.dot(
                vnf_ref[:, c * h_dim:(c + 1) * h_dim], we1v_ref[...],
                preferred_element_type=f32) + be1_ref[...]).astype(bf16)
        vscr_ref[:, ch:ch + 3 * c_dim] = vc_ref[...].astype(bf16)

    db = db_ref[0, 0, :]
    onehot = (db[:, None] == jax.lax.broadcasted_iota(
        jnp.int32, (s, b_dim), 1)).astype(bf16)

    onehot_t = (db[None, :] == jax.lax.broadcasted_iota(
        jnp.int32, (b_dim, s), 0)).astype(bf16)

    def scat(rhs):
        return jnp.dot(onehot_t, rhs, preferred_element_type=f32)

    gat = jnp.dot(onehot, vscr_ref[...], preferred_element_type=f32)

    crd = crd_ref[...]
    crd_rep = jnp.concatenate(
        [crd[:, k:k + 1] for k in range(3) for _ in range(c_dim)], axis=1)
    d12 = gat[:, ch:ch + 3 * c_dim] - crd_rep

    rad_all = jnp.sqrt(jnp.dot((d12 * d12).astype(bf16), sel_ref[...],
                               preferred_element_type=f32))

    nfw = jnp.dot(nf_ref[...].astype(bf16), we1n_ref[...].astype(bf16),
                  preferred_element_type=f32)

    we1r = we1r_ref[...]
    we2 = we2_ref[...].astype(bf16)
    wc1 = wc1_ref[...].astype(bf16)
    cms = []
    for c in range(c_dim):
        pre1 = (nfw + gat[:, c * h_dim:(c + 1) * h_dim]
                + rad_all[:, c:c + 1] * we1r)
        h1 = _silu(pre1).astype(bf16)
        h2 = _silu(jnp.dot(h1, we2, preferred_element_type=f32)
                   + be2_ref[...])
        h2b = h2.astype(bf16)
        accn_ref[:, c * h_dim:(c + 1) * h_dim] += scat(h2b)
        cms.append(_silu(jnp.dot(h2b, wc1, preferred_element_type=f32)
                         + bc1_ref[...]).astype(bf16))

    s4 = jnp.dot(jnp.concatenate(cms, axis=1), wc2blk_ref[...],
                 preferred_element_type=f32)
    s12 = jnp.concatenate([s4, s4, s4], axis=1)
    ones = jnp.ones((s, 1), dtype=bf16)
    trans = jnp.concatenate([(d12 * s12).astype(bf16), ones], axis=1)
    accc_ref[...] += scat(trans)

    @pl.when(i == nb - 1)
    def _epilogue():
        cnt = jnp.maximum(accc_ref[:, 3 * c_dim:3 * c_dim + 1], 1.0)
        inv = 1.0 / cnt
        outc_ref[...] = vc_ref[...] + accc_ref[:, :3 * c_dim] * inv
        for c in range(c_dim):
            vnf_c = vnf_ref[:, c * h_dim:(c + 1) * h_dim]
            agg_c = accn_ref[:, c * h_dim:(c + 1) * h_dim] * inv
            t = _silu(jnp.dot(vnf_c, wn1a_ref[...], preferred_element_type=f32)
                      + jnp.dot(agg_c, wn1b_ref[...], preferred_element_type=f32)
                      + bn1_ref[...])
            o = (jnp.dot(t, wn2_ref[...], preferred_element_type=f32)
                 + bn2_ref[...] + vnf_c)
            outf_ref[:, c * h_dim:(c + 1) * h_dim] = o


def kernel(node_feat, coord, virtual_node_feat, virtual_coord,
           We1, be1, We2, be2, Wc1, bc1, Wc2, Wn1, bn1, Wn2, bn2,
           data_batch):
    n, nf = node_feat.shape
    b, h, c = virtual_node_feat.shape
    ch = c * h

    s = 5000
    nb = -(-n // s)
    n_pad = nb * s
    if n_pad != n:
        node_feat = jnp.pad(node_feat, ((0, n_pad - n), (0, 0)))
        coord = jnp.pad(coord, ((0, n_pad - n), (0, 0)))
        data_batch = jnp.pad(data_batch, (0, n_pad - n),
                             constant_values=jnp.int32(b))

    vnf_t = virtual_node_feat.transpose(0, 2, 1).reshape(b, ch)
    vc_flat = virtual_coord.reshape(b, 3 * c)
    db3 = data_batch.reshape(nb, 1, s)

    we1_nf = We1[:nf]
    we1_v = We1[nf:nf + h]
    we1_r = We1[2 * nf:2 * nf + 1]
    wn1a = Wn1[:h]
    wn1b = Wn1[h:]

    row = lambda v: v.reshape(1, -1)

    grid_spec = pltpu.PrefetchScalarGridSpec(
        num_scalar_prefetch=0,
        grid=(nb,),
        in_specs=[
            pl.BlockSpec((s, nf), lambda i: (i, 0)),
            pl.BlockSpec((s, 3), lambda i: (i, 0)),
            pl.BlockSpec((1, 1, s), lambda i: (i, 0, 0)),
            pl.BlockSpec((b, ch), lambda i: (0, 0)),
            pl.BlockSpec((b, 3 * c), lambda i: (0, 0)),
            pl.BlockSpec((nf, h), lambda i: (0, 0)),
            pl.BlockSpec((h, h), lambda i: (0, 0)),
            pl.BlockSpec((1, h), lambda i: (0, 0)),
            pl.BlockSpec((1, h), lambda i: (0, 0)),
            pl.BlockSpec((h, h), lambda i: (0, 0)),
            pl.BlockSpec((1, h), lambda i: (0, 0)),
            pl.BlockSpec((h, h), lambda i: (0, 0)),
            pl.BlockSpec((1, h), lambda i: (0, 0)),
            pl.BlockSpec((h, 1), lambda i: (0, 0)),
            pl.BlockSpec((h, h), lambda i: (0, 0)),
            pl.BlockSpec((h, h), lambda i: (0, 0)),
            pl.BlockSpec((1, h), lambda i: (0, 0)),
            pl.BlockSpec((h, h), lambda i: (0, 0)),
            pl.BlockSpec((1, h), lambda i: (0, 0)),
            pl.BlockSpec((3 * c, c), lambda i: (0, 0)),
            pl.BlockSpec((ch, c), lambda i: (0, 0)),
        ],
        out_specs=[
            pl.BlockSpec((b, ch), lambda i: (0, 0)),
            pl.BlockSpec((b, 3 * c), lambda i: (0, 0)),
        ],
        scratch_shapes=[
            pltpu.VMEM((b, ch), jnp.float32),
            pltpu.VMEM((b, 3 * c + 1), jnp.float32),
            pltpu.VMEM((b, ch + 3 * c), jnp.bfloat16),
        ],
    )

    call = pl.pallas_call(
        functools.partial(_egcl_block_kernel, nb=nb, c_dim=c, h_dim=h),
        grid_spec=grid_spec,
        out_shape=[
            jax.ShapeDtypeStruct((b, ch), jnp.float32),
            jax.ShapeDtypeStruct((b, 3 * c), jnp.float32),
        ],
        compiler_params=pltpu.CompilerParams(
            dimension_semantics=("arbitrary",)),
    )

    sel = jnp.tile(jnp.eye(c, dtype=jnp.bfloat16), (3, 1))
    wc2blk = jnp.kron(jnp.eye(c, dtype=jnp.float32),
                      Wc2).astype(jnp.bfloat16)

    outf, outc = call(node_feat, coord, db3, vnf_t, vc_flat,
                      we1_nf, we1_v, we1_r, row(be1), We2, row(be2),
                      Wc1, row(bc1), Wc2, wn1a, wn1b, row(bn1), Wn2,
                      row(bn2), sel, wc2blk)

    new_vfeat = outf.reshape(b, c, h).transpose(0, 2, 1)
    new_vcoord = outc.reshape(b, 3, c)
    return (new_vfeat, new_vcoord)


# speedup vs baseline: 11.7620x; 1.1945x over previous
import functools

import jax
import jax.numpy as jnp
from jax.experimental import pallas as pl
from jax.experimental.pallas import tpu as pltpu


def _silu_h(m):
    return m * (jnp.tanh(m) + 1.0)


def _egcl_block_kernel(nf_ref, crd_ref, db_ref, vnf_ref, vc_ref,
                       we1n_ref, we1v_ref, we1r_ref, be1_ref,
                       we2_ref, be2_ref, wc1_ref, bc1_ref, wc2_ref,
                       wn1a_ref, wn1b_ref, bn1_ref, wn2_ref, bn2_ref,
                       sel_ref, wc2blk_ref,
                       outf_ref, outc_ref,
                       accn_ref, accc_ref, vscr_ref,
                       *, nb, c_dim, h_dim):
    i = pl.program_id(0)
    b_dim = vnf_ref.shape[0]
    s = nf_ref.shape[0]
    ch = c_dim * h_dim
    f32 = jnp.float32
    bf16 = jnp.bfloat16

    @pl.when(i == 0)
    def _init():
        accn_ref[...] = jnp.zeros_like(accn_ref)
        accc_ref[...] = jnp.zeros_like(accc_ref)
        for c in range(c_dim):
            vscr_ref[:, c * h_dim:(c + 1) * h_dim] = (jnp.dot(
                vnf_ref[:, c * h_dim:(c + 1) * h_dim], we1v_ref[...],
                preferred_element_type=f32) + be1_ref[...]).astype(bf16)
        vscr_ref[:, ch:ch + 3 * c_dim] = vc_ref[...].astype(bf16)

    db = db_ref[0, 0, :]
    onehot = (db[:, None] == jax.lax.broadcasted_iota(
        jnp.int32, (s, b_dim), 1)).astype(bf16)

    onehot_t = (db[None, :] == jax.lax.broadcasted_iota(
        jnp.int32, (b_dim, s), 0)).astype(bf16)

    def scat(rhs):
        return jnp.dot(onehot_t, rhs, preferred_element_type=f32)

    gat = jnp.dot(onehot, vscr_ref[...], preferred_element_type=f32)

    d12 = gat[:, ch:ch + 3 * c_dim] - crd_ref[...]

    rad_all = jnp.sqrt(jnp.dot((d12 * d12).astype(bf16), sel_ref[...],
                               preferred_element_type=f32))

    nfw = jnp.dot(nf_ref[...].astype(bf16), we1n_ref[...].astype(bf16),
                  preferred_element_type=f32)

    we1r = we1r_ref[...]
    we2 = we2_ref[...].astype(bf16)
    wc1 = wc1_ref[...].astype(bf16)
    cms = []
    for c in range(c_dim):
        pre1 = (nfw + gat[:, c * h_dim:(c + 1) * h_dim]
                + rad_all[:, c:c + 1] * we1r)
        h1 = _silu_h(pre1).astype(bf16)
        h2 = _silu_h(jnp.dot(h1, we2, preferred_element_type=f32)
                     + be2_ref[...])
        h2b = h2.astype(bf16)
        accn_ref[:, c * h_dim:(c + 1) * h_dim] += scat(h2b)
        cms.append(_silu_h(jnp.dot(h2b, wc1, preferred_element_type=f32)
                           + bc1_ref[...]).astype(bf16))

    s4 = jnp.dot(jnp.concatenate(cms, axis=1), wc2blk_ref[...],
                 preferred_element_type=f32)
    s12 = jnp.concatenate([s4, s4, s4], axis=1)
    ones = jnp.ones((s, 1), dtype=bf16)
    trans = jnp.concatenate([(d12 * s12).astype(bf16), ones], axis=1)
    accc_ref[...] += scat(trans)

    @pl.when(i == nb - 1)
    def _epilogue():
        cnt = jnp.maximum(accc_ref[:, 3 * c_dim:3 * c_dim + 1], 1.0)
        inv = 1.0 / cnt
        outc_ref[...] = vc_ref[...] + accc_ref[:, :3 * c_dim] * inv
        for c in range(c_dim):
            vnf_c = vnf_ref[:, c * h_dim:(c + 1) * h_dim]
            agg_c = accn_ref[:, c * h_dim:(c + 1) * h_dim] * inv
            t = _silu_h(jnp.dot(vnf_c, wn1a_ref[...], preferred_element_type=f32)
                        + jnp.dot(agg_c, wn1b_ref[...], preferred_element_type=f32)
                        + bn1_ref[...])
            o = (jnp.dot(t, wn2_ref[...], preferred_element_type=f32)
                 + bn2_ref[...] + vnf_c)
            outf_ref[:, c * h_dim:(c + 1) * h_dim] = o


def kernel(node_feat, coord, virtual_node_feat, virtual_coord,
           We1, be1, We2, be2, Wc1, bc1, Wc2, Wn1, bn1, Wn2, bn2,
           data_batch):
    n, nf = node_feat.shape
    b, h, c = virtual_node_feat.shape
    ch = c * h

    s = 5000
    nb = -(-n // s)
    n_pad = nb * s
    coord_rep = jnp.repeat(coord, c, axis=1)
    if n_pad != n:
        node_feat = jnp.pad(node_feat, ((0, n_pad - n), (0, 0)))
        coord_rep = jnp.pad(coord_rep, ((0, n_pad - n), (0, 0)))
        data_batch = jnp.pad(data_batch, (0, n_pad - n),
                             constant_values=jnp.int32(b))

    vnf_t = virtual_node_feat.transpose(0, 2, 1).reshape(b, ch)
    vc_flat = virtual_coord.reshape(b, 3 * c)
    db3 = data_batch.reshape(nb, 1, s)

    we1_nf = We1[:nf] * 0.5
    we1_v = We1[nf:nf + h] * 0.5
    we1_r = We1[2 * nf:2 * nf + 1] * 0.5
    be1 = be1 * 0.5
    We2 = We2 * 0.5
    be2 = be2 * 0.5
    Wc1 = Wc1 * 0.5
    bc1 = bc1 * 0.5
    wn1a = Wn1[:h] * 0.5
    wn1b = Wn1[h:] * 0.5
    bn1 = bn1 * 0.5

    row = lambda v: v.reshape(1, -1)

    grid_spec = pltpu.PrefetchScalarGridSpec(
        num_scalar_prefetch=0,
        grid=(nb,),
        in_specs=[
            pl.BlockSpec((s, nf), lambda i: (i, 0)),
            pl.BlockSpec((s, 3 * c), lambda i: (i, 0)),
            pl.BlockSpec((1, 1, s), lambda i: (i, 0, 0)),
            pl.BlockSpec((b, ch), lambda i: (0, 0)),
            pl.BlockSpec((b, 3 * c), lambda i: (0, 0)),
            pl.BlockSpec((nf, h), lambda i: (0, 0)),
            pl.BlockSpec((h, h), lambda i: (0, 0)),
            pl.BlockSpec((1, h), lambda i: (0, 0)),
            pl.BlockSpec((1, h), lambda i: (0, 0)),
            pl.BlockSpec((h, h), lambda i: (0, 0)),
            pl.BlockSpec((1, h), lambda i: (0, 0)),
            pl.BlockSpec((h, h), lambda i: (0, 0)),
            pl.BlockSpec((1, h), lambda i: (0, 0)),
            pl.BlockSpec((h, 1), lambda i: (0, 0)),
            pl.BlockSpec((h, h), lambda i: (0, 0)),
            pl.BlockSpec((h, h), lambda i: (0, 0)),
            pl.BlockSpec((1, h), lambda i: (0, 0)),
            pl.BlockSpec((h, h), lambda i: (0, 0)),
            pl.BlockSpec((1, h), lambda i: (0, 0)),
            pl.BlockSpec((3 * c, c), lambda i: (0, 0)),
            pl.BlockSpec((ch, c), lambda i: (0, 0)),
        ],
        out_specs=[
            pl.BlockSpec((b, ch), lambda i: (0, 0)),
            pl.BlockSpec((b, 3 * c), lambda i: (0, 0)),
        ],
        scratch_shapes=[
            pltpu.VMEM((b, ch), jnp.float32),
            pltpu.VMEM((b, 3 * c + 1), jnp.float32),
            pltpu.VMEM((b, ch + 3 * c), jnp.bfloat16),
        ],
    )

    call = pl.pallas_call(
        functools.partial(_egcl_block_kernel, nb=nb, c_dim=c, h_dim=h),
        grid_spec=grid_spec,
        out_shape=[
            jax.ShapeDtypeStruct((b, ch), jnp.float32),
            jax.ShapeDtypeStruct((b, 3 * c), jnp.float32),
        ],
        compiler_params=pltpu.CompilerParams(
            dimension_semantics=("arbitrary",)),
    )

    sel = jnp.tile(jnp.eye(c, dtype=jnp.bfloat16), (3, 1))
    wc2blk = jnp.kron(jnp.eye(c, dtype=jnp.float32),
                      Wc2).astype(jnp.bfloat16)

    outf, outc = call(node_feat, coord_rep, db3, vnf_t, vc_flat,
                      we1_nf, we1_v, we1_r, row(be1), We2, row(be2),
                      Wc1, row(bc1), Wc2, wn1a, wn1b, row(bn1), Wn2,
                      row(bn2), sel, wc2blk)

    new_vfeat = outf.reshape(b, c, h).transpose(0, 2, 1)
    new_vcoord = outc.reshape(b, 3, c)
    return (new_vfeat, new_vcoord)


# speedup vs baseline: 11.8166x; 1.0046x over previous
import functools

import jax
import jax.numpy as jnp
from jax.experimental import pallas as pl
from jax.experimental.pallas import tpu as pltpu


def _silu_h(m):
    return m * (jnp.tanh(m) + 1.0)


def _egcl_block_kernel(nf_ref, crd_ref, db_ref, vnf_ref, vc_ref,
                       we1n_ref, we1v_ref, we1r_ref, be1_ref,
                       we2_ref, be2_ref, wc1_ref, bc1_ref, wc2_ref,
                       wn1a_ref, wn1b_ref, bn1_ref, wn2_ref, bn2_ref,
                       sel_ref, wc2blk_ref,
                       outf_ref, outc_ref,
                       accn_ref, accc_ref, vscr_ref,
                       *, nb, c_dim, h_dim):
    i = pl.program_id(0)
    b_dim = vnf_ref.shape[0]
    s = nf_ref.shape[0]
    ch = c_dim * h_dim
    f32 = jnp.float32
    bf16 = jnp.bfloat16

    @pl.when(i == 0)
    def _init():
        accn_ref[...] = jnp.zeros_like(accn_ref)
        accc_ref[...] = jnp.zeros_like(accc_ref)
        for c in range(c_dim):
            vscr_ref[:, c * h_dim:(c + 1) * h_dim] = (jnp.dot(
                vnf_ref[:, c * h_dim:(c + 1) * h_dim],
                we1v_ref[...] * 0.5,
                preferred_element_type=f32) + be1_ref[...] * 0.5).astype(bf16)
        vscr_ref[:, ch:ch + 3 * c_dim] = vc_ref[...].astype(bf16)

    db = db_ref[0, 0, :]
    onehot = (db[:, None] == jax.lax.broadcasted_iota(
        jnp.int32, (s, b_dim), 1)).astype(bf16)

    onehot_t = (db[None, :] == jax.lax.broadcasted_iota(
        jnp.int32, (b_dim, s), 0)).astype(bf16)

    def scat(rhs):
        return jnp.dot(onehot_t, rhs, preferred_element_type=f32)

    gat = jnp.dot(onehot, vscr_ref[...], preferred_element_type=f32)

    d12 = gat[:, ch:ch + 3 * c_dim] - crd_ref[...]

    rad_all = jnp.sqrt(jnp.dot((d12 * d12).astype(bf16), sel_ref[...],
                               preferred_element_type=f32))

    nfw = jnp.dot(nf_ref[...].astype(bf16),
                  (we1n_ref[...] * 0.5).astype(bf16),
                  preferred_element_type=f32)

    we1r = we1r_ref[...] * 0.5
    be2 = be2_ref[...] * 0.5
    bc1 = bc1_ref[...] * 0.5
    we2 = (we2_ref[...] * 0.5).astype(bf16)
    wc1 = (wc1_ref[...] * 0.5).astype(bf16)
    cms = []
    for c in range(c_dim):
        pre1 = (nfw + gat[:, c * h_dim:(c + 1) * h_dim]
                + rad_all[:, c:c + 1] * we1r)
        h1 = _silu_h(pre1).astype(bf16)
        h2 = _silu_h(jnp.dot(h1, we2, preferred_element_type=f32)
                     + be2)
        h2b = h2.astype(bf16)
        accn_ref[:, c * h_dim:(c + 1) * h_dim] += scat(h2b)
        cms.append(_silu_h(jnp.dot(h2b, wc1, preferred_element_type=f32)
                           + bc1).astype(bf16))

    s4 = jnp.dot(jnp.concatenate(cms, axis=1), wc2blk_ref[...],
                 preferred_element_type=f32)
    s12 = jnp.concatenate([s4, s4, s4], axis=1)
    ones = jnp.ones((s, 1), dtype=bf16)
    trans = jnp.concatenate([(d12 * s12).astype(bf16), ones], axis=1)
    accc_ref[...] += scat(trans)

    @pl.when(i == nb - 1)
    def _epilogue():
        cnt = jnp.maximum(accc_ref[:, 3 * c_dim:3 * c_dim + 1], 1.0)
        inv = 1.0 / cnt
        outc_ref[...] = vc_ref[...] + accc_ref[:, :3 * c_dim] * inv
        for c in range(c_dim):
            vnf_c = vnf_ref[:, c * h_dim:(c + 1) * h_dim]
            agg_c = accn_ref[:, c * h_dim:(c + 1) * h_dim] * inv
            t = _silu_h(jnp.dot(vnf_c, wn1a_ref[...] * 0.5,
                                preferred_element_type=f32)
                        + jnp.dot(agg_c, wn1b_ref[...] * 0.5,
                                  preferred_element_type=f32)
                        + bn1_ref[...] * 0.5)
            o = (jnp.dot(t, wn2_ref[...], preferred_element_type=f32)
                 + bn2_ref[...] + vnf_c)
            outf_ref[:, c * h_dim:(c + 1) * h_dim] = o


def kernel(node_feat, coord, virtual_node_feat, virtual_coord,
           We1, be1, We2, be2, Wc1, bc1, Wc2, Wn1, bn1, Wn2, bn2,
           data_batch):
    n, nf = node_feat.shape
    b, h, c = virtual_node_feat.shape
    ch = c * h

    s = 5000
    nb = -(-n // s)
    n_pad = nb * s
    coord_rep = jnp.repeat(coord, c, axis=1)
    if n_pad != n:
        node_feat = jnp.pad(node_feat, ((0, n_pad - n), (0, 0)))
        coord_rep = jnp.pad(coord_rep, ((0, n_pad - n), (0, 0)))
        data_batch = jnp.pad(data_batch, (0, n_pad - n),
                             constant_values=jnp.int32(b))

    vnf_t = virtual_node_feat.transpose(0, 2, 1).reshape(b, ch)
    vc_flat = virtual_coord.reshape(b, 3 * c)
    db3 = data_batch.reshape(nb, 1, s)

    we1_nf = We1[:nf]
    we1_v = We1[nf:nf + h]
    we1_r = We1[2 * nf:2 * nf + 1]
    wn1a = Wn1[:h]
    wn1b = Wn1[h:]

    row = lambda v: v.reshape(1, -1)

    grid_spec = pltpu.PrefetchScalarGridSpec(
        num_scalar_prefetch=0,
        grid=(nb,),
        in_specs=[
            pl.BlockSpec((s, nf), lambda i: (i, 0)),
            pl.BlockSpec((s, 3 * c), lambda i: (i, 0)),
            pl.BlockSpec((1, 1, s), lambda i: (i, 0, 0)),
            pl.BlockSpec((b, ch), lambda i: (0, 0)),
            pl.BlockSpec((b, 3 * c), lambda i: (0, 0)),
            pl.BlockSpec((nf, h), lambda i: (0, 0)),
            pl.BlockSpec((h, h), lambda i: (0, 0)),
            pl.BlockSpec((1, h), lambda i: (0, 0)),
            pl.BlockSpec((1, h), lambda i: (0, 0)),
            pl.BlockSpec((h, h), lambda i: (0, 0)),
            pl.BlockSpec((1, h), lambda i: (0, 0)),
            pl.BlockSpec((h, h), lambda i: (0, 0)),
            pl.BlockSpec((1, h), lambda i: (0, 0)),
            pl.BlockSpec((h, 1), lambda i: (0, 0)),
            pl.BlockSpec((h, h), lambda i: (0, 0)),
            pl.BlockSpec((h, h), lambda i: (0, 0)),
            pl.BlockSpec((1, h), lambda i: (0, 0)),
            pl.BlockSpec((h, h), lambda i: (0, 0)),
            pl.BlockSpec((1, h), lambda i: (0, 0)),
            pl.BlockSpec((3 * c, c), lambda i: (0, 0)),
            pl.BlockSpec((ch, c), lambda i: (0, 0)),
        ],
        out_specs=[
            pl.BlockSpec((b, ch), lambda i: (0, 0)),
            pl.BlockSpec((b, 3 * c), lambda i: (0, 0)),
        ],
        scratch_shapes=[
            pltpu.VMEM((b, ch), jnp.float32),
            pltpu.VMEM((b, 3 * c + 1), jnp.float32),
            pltpu.VMEM((b, ch + 3 * c), jnp.bfloat16),
        ],
    )

    call = pl.pallas_call(
        functools.partial(_egcl_block_kernel, nb=nb, c_dim=c, h_dim=h),
        grid_spec=grid_spec,
        out_shape=[
            jax.ShapeDtypeStruct((b, ch), jnp.float32),
            jax.ShapeDtypeStruct((b, 3 * c), jnp.float32),
        ],
        compiler_params=pltpu.CompilerParams(
            dimension_semantics=("arbitrary",)),
    )

    sel = jnp.tile(jnp.eye(c, dtype=jnp.bfloat16), (3, 1))
    wc2blk = jnp.kron(jnp.eye(c, dtype=jnp.float32),
                      Wc2).astype(jnp.bfloat16)

    outf, outc = call(node_feat, coord_rep, db3, vnf_t, vc_flat,
                      we1_nf, we1_v, we1_r, row(be1), We2, row(be2),
                      Wc1, row(bc1), Wc2, wn1a, wn1b, row(bn1), Wn2,
                      row(bn2), sel, wc2blk)

    new_vfeat = outf.reshape(b, c, h).transpose(0, 2, 1)
    new_vcoord = outc.reshape(b, 3, c)
    return (new_vfeat, new_vcoord)
